# baseline (speedup 1.0000x reference)
import functools

import jax
import jax.numpy as jnp
from jax import lax
from jax.experimental import pallas as pl
from jax.experimental.pallas import tpu as pltpu
from jax.experimental.pallas import tpu_sc as plsc


def _make_gather(V, D, B):
    info = plsc.get_sparse_core_info()
    NC, NS = info.num_cores, info.num_subcores
    NW = NC * NS
    assert B % NW == 0
    b_per_w = B // NW
    CHUNK = 128
    assert b_per_w % CHUNK == 0
    n_chunks = b_per_w // CHUNK
    NBUF = 2
    assert n_chunks % NBUF == 0

    mesh = plsc.VectorSubcoreMesh(core_axis_name="c", subcore_axis_name="s")

    @functools.partial(
        pl.kernel,
        mesh=mesh,
        compiler_params=pltpu.CompilerParams(use_tc_tiling_on_sc=False),
        out_type=jax.ShapeDtypeStruct((B, D), jnp.float32),
        scratch_types=[
            pltpu.VMEM((n_chunks, CHUNK), jnp.int32),
            pltpu.VMEM((CHUNK, D), jnp.float32),
            pltpu.VMEM((CHUNK, D), jnp.float32),
            pltpu.SemaphoreType.DMA,
            pltpu.SemaphoreType.DMA,
        ],
    )
    def k(idx_hbm, table_hbm, out_hbm, idx_v, buf0, buf1, sem0, sem1):
        wid = lax.axis_index("s") * NC + lax.axis_index("c")
        base = wid * b_per_w
        pltpu.sync_copy(idx_hbm.at[wid], idx_v)
        bufs = (buf0, buf1)
        sems = (sem0, sem1)
        for b in range(NBUF):
            pltpu.async_copy(table_hbm.at[idx_v.at[b]], bufs[b], sems[b])

        def body(grp, _):
            j0 = grp * NBUF
            for b in range(NBUF):
                j = j0 + b
                pltpu.make_async_copy(
                    table_hbm.at[idx_v.at[j]], bufs[b], sems[b]
                ).wait()
                pltpu.sync_copy(
                    bufs[b], out_hbm.at[pl.ds(base + j * CHUNK, CHUNK)]
                )
                nxt = j + NBUF

                @pl.when(nxt < n_chunks)
                def _():
                    pltpu.async_copy(
                        table_hbm.at[idx_v.at[nxt]], bufs[b], sems[b]
                    )
            return _

        lax.fori_loop(0, n_chunks // NBUF, body, None)

    return k


def kernel(inputs, table):
    S, Bt = inputs.shape
    V, D = table.shape
    B = S * Bt
    info = plsc.get_sparse_core_info()
    NW = info.num_cores * info.num_subcores
    b_per_w = B // NW
    CHUNK = 128
    n_chunks = b_per_w // CHUNK
    idx = inputs.astype(jnp.int32).reshape(NW, n_chunks, CHUNK)
    out = _make_gather(V, D, B)(idx, table)
    return out.reshape(S, Bt, D)


# speedup vs baseline: 1.0063x; 1.0063x over previous
import functools

import jax
import jax.numpy as jnp
from jax import lax
from jax.experimental import pallas as pl
from jax.experimental.pallas import tpu as pltpu
from jax.experimental.pallas import tpu_sc as plsc


def _make_gather(V, D, B):
    info = plsc.get_sparse_core_info()
    NC, NS = info.num_cores, info.num_subcores
    NW = NC * NS
    assert B % NW == 0
    b_per_w = B // NW
    CHUNK = 128
    assert b_per_w % CHUNK == 0
    n_chunks = b_per_w // CHUNK
    NBUF = 10
    PRE = 5
    assert n_chunks % NBUF == 0

    mesh = plsc.VectorSubcoreMesh(core_axis_name="c", subcore_axis_name="s")

    @functools.partial(
        pl.kernel,
        mesh=mesh,
        compiler_params=pltpu.CompilerParams(use_tc_tiling_on_sc=False),
        out_type=jax.ShapeDtypeStruct((B, D), jnp.float32),
        scratch_types=[
            pltpu.VMEM((n_chunks, CHUNK), jnp.int32),
            [pltpu.VMEM((CHUNK, D), jnp.float32) for _ in range(NBUF)],
            [pltpu.SemaphoreType.DMA for _ in range(NBUF)],
            [pltpu.SemaphoreType.DMA for _ in range(NBUF)],
        ],
    )
    def k(idx_hbm, table_hbm, out_hbm, idx_v, bufs, sem_g, sem_w):
        wid = lax.axis_index("s") * NC + lax.axis_index("c")
        base = wid * b_per_w
        pltpu.sync_copy(idx_hbm.at[wid], idx_v)

        def gather(j, b):
            pltpu.async_copy(table_hbm.at[idx_v.at[j]], bufs[b], sems_g[b])

        sems_g, sems_w = sem_g, sem_w
        for j in range(PRE):
            gather(j, j % NBUF)

        def body(grp, _):
            j0 = grp * NBUF
            for b in range(NBUF):
                j = j0 + b
                pltpu.make_async_copy(
                    table_hbm.at[idx_v.at[j]], bufs[b], sems_g[b]
                ).wait()
                pltpu.async_copy(
                    bufs[b], out_hbm.at[pl.ds(base + j * CHUNK, CHUNK)],
                    sems_w[b],
                )
                g = j + PRE
                bp = (b + PRE) % NBUF

                @pl.when(jnp.logical_and(g < n_chunks, g >= NBUF))
                def _():
                    pltpu.make_async_copy(
                        bufs[bp],
                        out_hbm.at[pl.ds(base + j * CHUNK, CHUNK)],
                        sems_w[bp],
                    ).wait()

                @pl.when(g < n_chunks)
                def _():
                    gather(g, bp)
            return _

        lax.fori_loop(0, n_chunks // NBUF, body, None)

        for b in range(NBUF):
            pltpu.make_async_copy(
                bufs[b], out_hbm.at[pl.ds(base, CHUNK)], sems_w[b]
            ).wait()

    return k


def kernel(inputs, table):
    S, Bt = inputs.shape
    V, D = table.shape
    B = S * Bt
    info = plsc.get_sparse_core_info()
    NW = info.num_cores * info.num_subcores
    b_per_w = B // NW
    CHUNK = 128
    n_chunks = b_per_w // CHUNK
    idx = inputs.astype(jnp.int32).reshape(NW, n_chunks, CHUNK)
    out = _make_gather(V, D, B)(idx, table)
    return out.reshape(S, Bt, D)
